# baseline (speedup 1.0000x reference)
import functools

import jax
import jax.numpy as jnp
from jax import lax
from jax.experimental import pallas as pl
from jax.experimental.pallas import tpu as pltpu
from jax.experimental.pallas import tpu_sc as plsc

NUM_EXPERTS = 8
TOP_K = 2
HIDDEN = 1024
MOE_INTER = 512
SHARED_INTER = 1024
T = 2048

BM = 128
NB = 40
R = NB * BM
NW = 32
TPW = T // NW


def _router_dispatch_body(x_ref, rw_ref, row1_ref, row2_ref,
                          row1s_ref, row2s_ref, w1_ref, w2_ref,
                          eblk_ref, nb_ref, xb_ref):
    x = x_ref[...]
    xb_ref[...] = x.astype(jnp.bfloat16)
    logits = lax.dot_general(x, rw_ref[...], (((1,), (1,)), ((), ())),
                             preferred_element_type=jnp.float32)
    probs = jax.nn.softmax(logits, axis=-1)
    e_iota = lax.broadcasted_iota(jnp.int32, probs.shape, 1)
    v1 = jnp.max(probs, axis=1, keepdims=True)
    i1 = jnp.argmax(probs, axis=1).reshape(-1, 1)
    masked = jnp.where(e_iota == i1, -jnp.inf, probs)
    v2 = jnp.max(masked, axis=1, keepdims=True)
    i2 = jnp.argmax(masked, axis=1).reshape(-1, 1)
    wsum = v1 + v2
    w1_ref[...] = jnp.broadcast_to(v1 / wsum, (T, 128))
    w2_ref[...] = jnp.broadcast_to(v2 / wsum, (T, 128))

    oh1 = (e_iota == i1).astype(jnp.bfloat16)
    oh2 = (e_iota == i2).astype(jnp.bfloat16)
    r_io = lax.broadcasted_iota(jnp.int32, (T, T), 0)
    c_io = lax.broadcasted_iota(jnp.int32, (T, T), 1)
    ltri = (c_io < r_io).astype(jnp.bfloat16)
    cum1 = lax.dot_general(ltri, oh1, (((1,), (0,)), ((), ())),
                           preferred_element_type=jnp.float32)
    cum2 = lax.dot_general(ltri, oh2, (((1,), (0,)), ((), ())),
                           preferred_element_type=jnp.float32)
    oh1f = oh1.astype(jnp.float32)
    oh2f = oh2.astype(jnp.float32)
    cnt1 = jnp.sum(oh1f, axis=0, keepdims=True)
    cnt2 = jnp.sum(oh2f, axis=0, keepdims=True)
    cnt = cnt1 + cnt2
    nb_e = jnp.floor((cnt + (BM - 1)) * (1.0 / BM))
    tri8 = (lax.broadcasted_iota(jnp.int32, (NUM_EXPERTS, NUM_EXPERTS), 0)
            <= lax.broadcasted_iota(jnp.int32, (NUM_EXPERTS, NUM_EXPERTS), 1)
            ).astype(jnp.float32)
    cumnb = lax.dot_general(nb_e, tri8, (((1,), (0,)), ((), ())),
                            preferred_element_type=jnp.float32)
    off = (cumnb - nb_e) * float(BM)
    row1 = jnp.sum(oh1f * (off + cum1), axis=1, keepdims=True)
    row2 = jnp.sum(oh2f * (off + cnt1 + cum2), axis=1, keepdims=True)
    row1_i = row1.astype(jnp.int32)
    row2_i = row2.astype(jnp.int32)
    row1_ref[...] = jnp.reshape(row1_i, (T // 64, 64))
    row2_ref[...] = jnp.reshape(row2_i, (T // 64, 64))
    row1s_ref[...] = jnp.reshape(row1_i, (T // 32, 32))
    row2s_ref[...] = jnp.reshape(row2_i, (T // 32, 32))

    cumnb_i = cumnb.astype(jnp.int32)
    iblk = lax.broadcasted_iota(jnp.int32, (NB, NUM_EXPERTS), 0)
    eblk = jnp.sum((iblk >= jnp.broadcast_to(cumnb_i, (NB, NUM_EXPERTS)))
                   .astype(jnp.int32), axis=1, keepdims=True)
    eblk = jnp.minimum(eblk, NUM_EXPERTS - 1)
    eblk_ref[...] = jnp.broadcast_to(eblk, (NB, 128))
    nblocks = cumnb_i[:, NUM_EXPERTS - 1:NUM_EXPERTS]
    nb_ref[...] = jnp.broadcast_to(nblocks, (8, 128))


def _shared_body(x_ref, sg_ref, su_ref, sd_ref, seg_ref, shared_out_ref):
    x = x_ref[...]
    xb = x.astype(jnp.bfloat16)
    g = lax.dot_general(xb, sg_ref[...].astype(jnp.bfloat16),
                        (((1,), (1,)), ((), ())),
                        preferred_element_type=jnp.float32)
    u = lax.dot_general(xb, su_ref[...].astype(jnp.bfloat16),
                        (((1,), (1,)), ((), ())),
                        preferred_element_type=jnp.float32)
    h = jax.nn.silu(g) * u
    shared = lax.dot_general(h.astype(jnp.bfloat16),
                             sd_ref[...].astype(jnp.bfloat16),
                             (((1,), (1,)), ((), ())),
                             preferred_element_type=jnp.float32)
    tok_gate = jax.nn.sigmoid(
        lax.dot_general(x, seg_ref[...], (((1,), (1,)), ((), ())),
                        preferred_element_type=jnp.float32))
    shared_out_ref[...] = tok_gate * shared


def _experts_sparse_body(pf_ref, xs_ref, gup_ref, dp_ref, wrow_ref, yw_ref):
    i = pl.program_id(0)
    nblocks = pf_ref[NB]

    @pl.when(i < nblocks)
    def _():
        xb = xs_ref[...]
        gu = lax.dot_general(xb, gup_ref[0].astype(jnp.bfloat16),
                             (((1,), (1,)), ((), ())),
                             preferred_element_type=jnp.float32)
        gate = gu[:, :MOE_INTER]
        up = gu[:, MOE_INTER:]
        h = jax.nn.silu(gate) * up
        y = lax.dot_general(h.astype(jnp.bfloat16),
                            dp_ref[0].astype(jnp.bfloat16),
                            (((1,), (1,)), ((), ())),
                            preferred_element_type=jnp.float32)
        yw_ref[...] = y * wrow_ref[...][:, 0:1]


def _sc_dispatch_body(x_hbm, row1_hbm, row2_hbm, w1_hbm, w2_hbm,
                      xs_hbm, wrow_hbm,
                      r1v, r2v, slab, wtmp1, wtmp2, sem1, sem2):
    wid = lax.axis_index("s") * 2 + lax.axis_index("c")
    base = wid * TPW
    pltpu.sync_copy(row1_hbm.at[pl.ds(wid, 1)], r1v)
    pltpu.sync_copy(row2_hbm.at[pl.ds(wid, 1)], r2v)
    pltpu.sync_copy(x_hbm.at[pl.ds(base, TPW)], slab)
    cp1 = pltpu.async_copy(slab, xs_hbm.at[r1v.at[0]], sem1)
    cp2 = pltpu.async_copy(slab, xs_hbm.at[r2v.at[0]], sem1)
    pltpu.sync_copy(w1_hbm.at[pl.ds(base, TPW)], wtmp1)
    pltpu.sync_copy(w2_hbm.at[pl.ds(base, TPW)], wtmp2)
    cp3 = pltpu.async_copy(wtmp1, wrow_hbm.at[r1v.at[0]], sem2)
    cp4 = pltpu.async_copy(wtmp2, wrow_hbm.at[r2v.at[0]], sem2)
    cp1.wait()
    cp2.wait()
    cp3.wait()
    cp4.wait()


def _sc_combine_body(yw_hbm, shared_hbm, row1_hbm, row2_hbm, out_hbm,
                     r1v, r2v, bufA, bufB, bufO, semA, semB):
    wid = lax.axis_index("s") * 2 + lax.axis_index("c")
    base = wid * TPW
    nchunks = TPW // 32
    pltpu.sync_copy(row1_hbm.at[pl.ds(wid * nchunks, nchunks)], r1v)
    pltpu.sync_copy(row2_hbm.at[pl.ds(wid * nchunks, nchunks)], r2v)
    for c in range(nchunks):
        tok0 = base + c * 32
        cpA = pltpu.async_copy(yw_hbm.at[r1v.at[c]], bufA, semA)
        cpB = pltpu.async_copy(yw_hbm.at[r2v.at[c]], bufB, semB)
        pltpu.sync_copy(shared_hbm.at[pl.ds(tok0, 32)], bufO)
        cpA.wait()
        cpB.wait()

        def row_body(t, carry):
            for v in range(HIDDEN // 16):
                sl = pl.ds(v * 16, 16)
                bufO[t, sl] = bufO[t, sl] + bufA[t, sl] + bufB[t, sl]
            return carry

        lax.fori_loop(0, 32, row_body, 0)
        pltpu.sync_copy(bufO, out_hbm.at[pl.ds(tok0, 32)])


_SC_KERNELS = []


def _get_sc_kernels():
    if not _SC_KERNELS:
        mesh = plsc.VectorSubcoreMesh(core_axis_name="c",
                                      subcore_axis_name="s")
        dispatch = pl.kernel(
            _sc_dispatch_body,
            mesh=mesh,
            out_type=[
                jax.ShapeDtypeStruct((R, HIDDEN // 2), jnp.int32),
                jax.ShapeDtypeStruct((R, 128), jnp.float32),
            ],
            scratch_types=[
                pltpu.VMEM((1, TPW), jnp.int32),
                pltpu.VMEM((1, TPW), jnp.int32),
                pltpu.VMEM((TPW, HIDDEN // 2), jnp.int32),
                pltpu.VMEM((TPW, 128), jnp.float32),
                pltpu.VMEM((TPW, 128), jnp.float32),
                pltpu.SemaphoreType.DMA,
                pltpu.SemaphoreType.DMA,
            ],
        )
        combine = pl.kernel(
            _sc_combine_body,
            mesh=mesh,
            out_type=jax.ShapeDtypeStruct((T, HIDDEN), jnp.float32),
            scratch_types=[
                pltpu.VMEM((TPW // 32, 32), jnp.int32),
                pltpu.VMEM((TPW // 32, 32), jnp.int32),
                pltpu.VMEM((32, HIDDEN), jnp.float32),
                pltpu.VMEM((32, HIDDEN), jnp.float32),
                pltpu.VMEM((32, HIDDEN), jnp.float32),
                pltpu.SemaphoreType.DMA,
                pltpu.SemaphoreType.DMA,
            ],
        )
        _SC_KERNELS.append((dispatch, combine))
    return _SC_KERNELS[0]


def _sc_dispatch(x, row1b, row2b, w1b, w2b):
    return _get_sc_kernels()[0](x, row1b, row2b, w1b, w2b)


def _sc_combine(yw, shared_out, row1b, row2b):
    return _get_sc_kernels()[1](yw, shared_out, row1b, row2b)


def kernel(hidden_states, router_weight, gate_up_proj, down_proj,
           shared_gate_proj, shared_up_proj, shared_down_proj,
           shared_expert_gate):
    B, S, H = hidden_states.shape
    x = hidden_states.reshape(-1, H)

    (row1b, row2b, row1s, row2s, w1b, w2b, eblk_b, nb_b,
     xb) = pl.pallas_call(
        _router_dispatch_body,
        in_specs=[
            pl.BlockSpec((T, H), lambda: (0, 0)),
            pl.BlockSpec((NUM_EXPERTS, H), lambda: (0, 0)),
        ],
        out_specs=[
            pl.BlockSpec((T // 64, 64), lambda: (0, 0)),
            pl.BlockSpec((T // 64, 64), lambda: (0, 0)),
            pl.BlockSpec((T // 32, 32), lambda: (0, 0)),
            pl.BlockSpec((T // 32, 32), lambda: (0, 0)),
            pl.BlockSpec((T, 128), lambda: (0, 0)),
            pl.BlockSpec((T, 128), lambda: (0, 0)),
            pl.BlockSpec((NB, 128), lambda: (0, 0)),
            pl.BlockSpec((8, 128), lambda: (0, 0)),
            pl.BlockSpec((T, HIDDEN), lambda: (0, 0)),
        ],
        out_shape=[
            jax.ShapeDtypeStruct((T // 64, 64), jnp.int32),
            jax.ShapeDtypeStruct((T // 64, 64), jnp.int32),
            jax.ShapeDtypeStruct((T // 32, 32), jnp.int32),
            jax.ShapeDtypeStruct((T // 32, 32), jnp.int32),
            jax.ShapeDtypeStruct((T, 128), jnp.float32),
            jax.ShapeDtypeStruct((T, 128), jnp.float32),
            jax.ShapeDtypeStruct((NB, 128), jnp.int32),
            jax.ShapeDtypeStruct((8, 128), jnp.int32),
            jax.ShapeDtypeStruct((T, HIDDEN), jnp.bfloat16),
        ],
    )(x, router_weight)

    pf = jnp.concatenate([eblk_b[:, 0], nb_b[0, 0:1]]).astype(jnp.int32)

    xb_i32 = lax.bitcast_convert_type(
        xb.reshape(T, H // 2, 2), jnp.int32)
    xs_i32, wrow = _sc_dispatch(xb_i32, row1b, row2b, w1b, w2b)
    xs = lax.bitcast_convert_type(xs_i32, jnp.bfloat16).reshape(R, H)

    BT = 1024
    shared_out = pl.pallas_call(
        _shared_body,
        grid=(T // BT,),
        in_specs=[
            pl.BlockSpec((BT, H), lambda i: (i, 0)),
            pl.BlockSpec((SHARED_INTER, H), lambda i: (0, 0)),
            pl.BlockSpec((SHARED_INTER, H), lambda i: (0, 0)),
            pl.BlockSpec((H, SHARED_INTER), lambda i: (0, 0)),
            pl.BlockSpec((1, H), lambda i: (0, 0)),
        ],
        out_specs=pl.BlockSpec((BT, H), lambda i: (i, 0)),
        out_shape=jax.ShapeDtypeStruct((T, H), jnp.float32),
        compiler_params=pltpu.CompilerParams(
            dimension_semantics=("arbitrary",)),
    )(x, shared_gate_proj, shared_up_proj, shared_down_proj,
      shared_expert_gate)

    grid_spec = pltpu.PrefetchScalarGridSpec(
        num_scalar_prefetch=1,
        grid=(NB,),
        in_specs=[
            pl.BlockSpec((BM, H), lambda i, pf: (i, 0)),
            pl.BlockSpec((1, 2 * MOE_INTER, H), lambda i, pf: (pf[i], 0, 0)),
            pl.BlockSpec((1, H, MOE_INTER), lambda i, pf: (pf[i], 0, 0)),
            pl.BlockSpec((BM, 128), lambda i, pf: (i, 0)),
        ],
        out_specs=pl.BlockSpec((BM, H), lambda i, pf: (i, 0)),
    )
    yw = pl.pallas_call(
        _experts_sparse_body,
        grid_spec=grid_spec,
        out_shape=jax.ShapeDtypeStruct((R, H), jnp.float32),
        compiler_params=pltpu.CompilerParams(
            dimension_semantics=("arbitrary",)),
    )(pf, xs, gate_up_proj, down_proj, wrow)

    out = _sc_combine(yw, shared_out, row1s, row2s)

    return out.reshape(B, S, H)


# speedup vs baseline: 2.1079x; 2.1079x over previous
import functools

import jax
import jax.numpy as jnp
from jax import lax
from jax.experimental import pallas as pl
from jax.experimental.pallas import tpu as pltpu
from jax.experimental.pallas import tpu_sc as plsc

NUM_EXPERTS = 8
TOP_K = 2
HIDDEN = 1024
MOE_INTER = 512
SHARED_INTER = 1024
T = 2048

BM = 128
NB = 40
R = NB * BM
NW = 32
TPW = T // NW


def _router_dispatch_body(x_ref, rw_ref, row1_ref, row2_ref,
                          row1s_ref, row2s_ref, w1_ref, w2_ref,
                          eblk_ref, nb_ref):
    x = x_ref[...]
    logits = lax.dot_general(x, rw_ref[...], (((1,), (1,)), ((), ())),
                             preferred_element_type=jnp.float32)
    probs = jax.nn.softmax(logits, axis=-1)
    e_iota = lax.broadcasted_iota(jnp.int32, probs.shape, 1)
    v1 = jnp.max(probs, axis=1, keepdims=True)
    i1 = jnp.argmax(probs, axis=1).reshape(-1, 1)
    masked = jnp.where(e_iota == i1, -jnp.inf, probs)
    v2 = jnp.max(masked, axis=1, keepdims=True)
    i2 = jnp.argmax(masked, axis=1).reshape(-1, 1)
    wsum = v1 + v2
    w1_ref[...] = jnp.broadcast_to(v1 / wsum, (T, 128))
    w2_ref[...] = jnp.broadcast_to(v2 / wsum, (T, 128))

    oh1 = (e_iota == i1).astype(jnp.bfloat16)
    oh2 = (e_iota == i2).astype(jnp.bfloat16)
    r_io = lax.broadcasted_iota(jnp.int32, (T, T), 0)
    c_io = lax.broadcasted_iota(jnp.int32, (T, T), 1)
    ltri = (c_io < r_io).astype(jnp.bfloat16)
    cum1 = lax.dot_general(ltri, oh1, (((1,), (0,)), ((), ())),
                           preferred_element_type=jnp.float32)
    cum2 = lax.dot_general(ltri, oh2, (((1,), (0,)), ((), ())),
                           preferred_element_type=jnp.float32)
    oh1f = oh1.astype(jnp.float32)
    oh2f = oh2.astype(jnp.float32)
    cnt1 = jnp.sum(oh1f, axis=0, keepdims=True)
    cnt2 = jnp.sum(oh2f, axis=0, keepdims=True)
    cnt = cnt1 + cnt2
    nb_e = jnp.floor((cnt + (BM - 1)) * (1.0 / BM))
    tri8 = (lax.broadcasted_iota(jnp.int32, (NUM_EXPERTS, NUM_EXPERTS), 0)
            <= lax.broadcasted_iota(jnp.int32, (NUM_EXPERTS, NUM_EXPERTS), 1)
            ).astype(jnp.float32)
    cumnb = lax.dot_general(nb_e, tri8, (((1,), (0,)), ((), ())),
                            preferred_element_type=jnp.float32)
    off = (cumnb - nb_e) * float(BM)
    row1 = jnp.sum(oh1f * (off + cum1), axis=1, keepdims=True)
    row2 = jnp.sum(oh2f * (off + cnt1 + cum2), axis=1, keepdims=True)
    row1_i = row1.astype(jnp.int32)
    row2_i = row2.astype(jnp.int32)
    row1_ref[...] = jnp.reshape(row1_i, (T // 64, 64))
    row2_ref[...] = jnp.reshape(row2_i, (T // 64, 64))
    row1s_ref[...] = jnp.reshape(row1_i, (T // 16, 16))
    row2s_ref[...] = jnp.reshape(row2_i, (T // 16, 16))

    cumnb_i = cumnb.astype(jnp.int32)
    iblk = lax.broadcasted_iota(jnp.int32, (NB, NUM_EXPERTS), 0)
    eblk = jnp.sum((iblk >= jnp.broadcast_to(cumnb_i, (NB, NUM_EXPERTS)))
                   .astype(jnp.int32), axis=1, keepdims=True)
    eblk = jnp.minimum(eblk, NUM_EXPERTS - 1)
    eblk_ref[...] = jnp.broadcast_to(eblk, (NB, 128))
    nblocks = cumnb_i[:, NUM_EXPERTS - 1:NUM_EXPERTS]
    nb_ref[...] = jnp.broadcast_to(nblocks, (8, 128))


def _shared_body(x_ref, sg_ref, su_ref, sd_ref, seg_ref, shared_out_ref):
    x = x_ref[...]
    xb = x.astype(jnp.bfloat16)
    g = lax.dot_general(xb, sg_ref[...].astype(jnp.bfloat16),
                        (((1,), (1,)), ((), ())),
                        preferred_element_type=jnp.float32)
    u = lax.dot_general(xb, su_ref[...].astype(jnp.bfloat16),
                        (((1,), (1,)), ((), ())),
                        preferred_element_type=jnp.float32)
    h = jax.nn.silu(g) * u
    shared = lax.dot_general(h.astype(jnp.bfloat16),
                             sd_ref[...].astype(jnp.bfloat16),
                             (((1,), (1,)), ((), ())),
                             preferred_element_type=jnp.float32)
    tok_gate = jax.nn.sigmoid(
        lax.dot_general(x, seg_ref[...], (((1,), (1,)), ((), ())),
                        preferred_element_type=jnp.float32))
    shared_out_ref[...] = tok_gate * shared


def _experts_sparse_body(pf_ref, xs_ref, gup_ref, dp_ref, wrow_ref, yw_ref):
    i = pl.program_id(0)
    nblocks = pf_ref[NB]

    @pl.when(i < nblocks)
    def _():
        xb = xs_ref[...].astype(jnp.bfloat16)
        gu = lax.dot_general(xb, gup_ref[0].astype(jnp.bfloat16),
                             (((1,), (1,)), ((), ())),
                             preferred_element_type=jnp.float32)
        gate = gu[:, :MOE_INTER]
        up = gu[:, MOE_INTER:]
        h = jax.nn.silu(gate) * up
        y = lax.dot_general(h.astype(jnp.bfloat16),
                            dp_ref[0].astype(jnp.bfloat16),
                            (((1,), (1,)), ((), ())),
                            preferred_element_type=jnp.float32)
        yw_ref[...] = y * wrow_ref[...][:, 0:1]


def _sc_dispatch_body(x_hbm, row1_hbm, row2_hbm, w1_hbm, w2_hbm,
                      xs_hbm, wrow_hbm,
                      r1v, r2v, slab, wtmp1, wtmp2, sem1, sem2):
    wid = lax.axis_index("s") * 2 + lax.axis_index("c")
    base = wid * TPW
    pltpu.sync_copy(row1_hbm.at[pl.ds(wid, 1)], r1v)
    pltpu.sync_copy(row2_hbm.at[pl.ds(wid, 1)], r2v)
    pltpu.sync_copy(x_hbm.at[pl.ds(base, TPW)], slab)
    cp1 = pltpu.async_copy(slab, xs_hbm.at[r1v.at[0]], sem1)
    cp2 = pltpu.async_copy(slab, xs_hbm.at[r2v.at[0]], sem1)
    pltpu.sync_copy(w1_hbm.at[pl.ds(base, TPW)], wtmp1)
    pltpu.sync_copy(w2_hbm.at[pl.ds(base, TPW)], wtmp2)
    cp3 = pltpu.async_copy(wtmp1, wrow_hbm.at[r1v.at[0]], sem2)
    cp4 = pltpu.async_copy(wtmp2, wrow_hbm.at[r2v.at[0]], sem2)
    cp1.wait()
    cp2.wait()
    cp3.wait()
    cp4.wait()


def _sc_combine_body(yw_hbm, shared_hbm, row1_hbm, row2_hbm, out_hbm,
                     r1v, r2v, bufA, bufB, bufO, semA, semB, semS):
    wid = lax.axis_index("s") * 2 + lax.axis_index("c")
    base = wid * TPW
    nchunks = TPW // 16
    pltpu.sync_copy(row1_hbm.at[pl.ds(wid * nchunks, nchunks)], r1v)
    pltpu.sync_copy(row2_hbm.at[pl.ds(wid * nchunks, nchunks)], r2v)

    def fire(c, p):
        tok0 = base + c * 16
        cps = [pltpu.async_copy(yw_hbm.at[r1v.at[c]], bufA.at[p], semA),
               pltpu.async_copy(yw_hbm.at[r2v.at[c]], bufB.at[p], semB),
               pltpu.async_copy(shared_hbm.at[pl.ds(tok0, 16)],
                                bufO.at[p], semS)]
        return cps

    pend = fire(0, 0)
    for c in range(nchunks):
        p = c % 2
        for cp in pend:
            cp.wait()
        if c + 1 < nchunks:
            pend = fire(c + 1, (c + 1) % 2)

        def row_body(t, carry):
            for v in range(HIDDEN // 16):
                sl = pl.ds(v * 16, 16)
                bufO[p, t, sl] = (bufO[p, t, sl] + bufA[p, t, sl]
                                  + bufB[p, t, sl])
            return carry

        lax.fori_loop(0, 16, row_body, 0)
        pltpu.sync_copy(bufO.at[p], out_hbm.at[pl.ds(base + c * 16, 16)])


_SC_KERNELS = []


def _get_sc_kernels():
    if not _SC_KERNELS:
        mesh = plsc.VectorSubcoreMesh(core_axis_name="c",
                                      subcore_axis_name="s")
        dispatch = pl.kernel(
            _sc_dispatch_body,
            mesh=mesh,
            out_type=[
                jax.ShapeDtypeStruct((R, HIDDEN), jnp.float32),
                jax.ShapeDtypeStruct((R, 128), jnp.float32),
            ],
            scratch_types=[
                pltpu.VMEM((1, TPW), jnp.int32),
                pltpu.VMEM((1, TPW), jnp.int32),
                pltpu.VMEM((TPW, HIDDEN), jnp.float32),
                pltpu.VMEM((TPW, 128), jnp.float32),
                pltpu.VMEM((TPW, 128), jnp.float32),
                pltpu.SemaphoreType.DMA,
                pltpu.SemaphoreType.DMA,
            ],
        )
        combine = pl.kernel(
            _sc_combine_body,
            mesh=mesh,
            out_type=jax.ShapeDtypeStruct((T, HIDDEN), jnp.float32),
            scratch_types=[
                pltpu.VMEM((TPW // 16, 16), jnp.int32),
                pltpu.VMEM((TPW // 16, 16), jnp.int32),
                pltpu.VMEM((2, 16, HIDDEN), jnp.float32),
                pltpu.VMEM((2, 16, HIDDEN), jnp.float32),
                pltpu.VMEM((2, 16, HIDDEN), jnp.float32),
                pltpu.SemaphoreType.DMA,
                pltpu.SemaphoreType.DMA,
                pltpu.SemaphoreType.DMA,
            ],
        )
        _SC_KERNELS.append((dispatch, combine))
    return _SC_KERNELS[0]


def _sc_dispatch(x, row1b, row2b, w1b, w2b):
    return _get_sc_kernels()[0](x, row1b, row2b, w1b, w2b)


def _sc_combine(yw, shared_out, row1b, row2b):
    return _get_sc_kernels()[1](yw, shared_out, row1b, row2b)


def kernel(hidden_states, router_weight, gate_up_proj, down_proj,
           shared_gate_proj, shared_up_proj, shared_down_proj,
           shared_expert_gate):
    B, S, H = hidden_states.shape
    x = hidden_states.reshape(-1, H)

    (row1b, row2b, row1s, row2s, w1b, w2b, eblk_b,
     nb_b) = pl.pallas_call(
        _router_dispatch_body,
        in_specs=[
            pl.BlockSpec((T, H), lambda: (0, 0)),
            pl.BlockSpec((NUM_EXPERTS, H), lambda: (0, 0)),
        ],
        out_specs=[
            pl.BlockSpec((T // 64, 64), lambda: (0, 0)),
            pl.BlockSpec((T // 64, 64), lambda: (0, 0)),
            pl.BlockSpec((T // 16, 16), lambda: (0, 0)),
            pl.BlockSpec((T // 16, 16), lambda: (0, 0)),
            pl.BlockSpec((T, 128), lambda: (0, 0)),
            pl.BlockSpec((T, 128), lambda: (0, 0)),
            pl.BlockSpec((NB, 128), lambda: (0, 0)),
            pl.BlockSpec((8, 128), lambda: (0, 0)),
        ],
        out_shape=[
            jax.ShapeDtypeStruct((T // 64, 64), jnp.int32),
            jax.ShapeDtypeStruct((T // 64, 64), jnp.int32),
            jax.ShapeDtypeStruct((T // 16, 16), jnp.int32),
            jax.ShapeDtypeStruct((T // 16, 16), jnp.int32),
            jax.ShapeDtypeStruct((T, 128), jnp.float32),
            jax.ShapeDtypeStruct((T, 128), jnp.float32),
            jax.ShapeDtypeStruct((NB, 128), jnp.int32),
            jax.ShapeDtypeStruct((8, 128), jnp.int32),
        ],
    )(x, router_weight)

    pf = jnp.concatenate([eblk_b[:, 0], nb_b[0, 0:1]]).astype(jnp.int32)

    xs, wrow = _sc_dispatch(x, row1b, row2b, w1b, w2b)

    BT = 1024
    shared_out = pl.pallas_call(
        _shared_body,
        grid=(T // BT,),
        in_specs=[
            pl.BlockSpec((BT, H), lambda i: (i, 0)),
            pl.BlockSpec((SHARED_INTER, H), lambda i: (0, 0)),
            pl.BlockSpec((SHARED_INTER, H), lambda i: (0, 0)),
            pl.BlockSpec((H, SHARED_INTER), lambda i: (0, 0)),
            pl.BlockSpec((1, H), lambda i: (0, 0)),
        ],
        out_specs=pl.BlockSpec((BT, H), lambda i: (i, 0)),
        out_shape=jax.ShapeDtypeStruct((T, H), jnp.float32),
        compiler_params=pltpu.CompilerParams(
            dimension_semantics=("arbitrary",)),
    )(x, shared_gate_proj, shared_up_proj, shared_down_proj,
      shared_expert_gate)

    grid_spec = pltpu.PrefetchScalarGridSpec(
        num_scalar_prefetch=1,
        grid=(NB,),
        in_specs=[
            pl.BlockSpec((BM, H), lambda i, pf: (i, 0)),
            pl.BlockSpec((1, 2 * MOE_INTER, H), lambda i, pf: (pf[i], 0, 0)),
            pl.BlockSpec((1, H, MOE_INTER), lambda i, pf: (pf[i], 0, 0)),
            pl.BlockSpec((BM, 128), lambda i, pf: (i, 0)),
        ],
        out_specs=pl.BlockSpec((BM, H), lambda i, pf: (i, 0)),
    )
    yw = pl.pallas_call(
        _experts_sparse_body,
        grid_spec=grid_spec,
        out_shape=jax.ShapeDtypeStruct((R, H), jnp.float32),
        compiler_params=pltpu.CompilerParams(
            dimension_semantics=("arbitrary",)),
    )(pf, xs, gate_up_proj, down_proj, wrow)

    out = _sc_combine(yw, shared_out, row1s, row2s)

    return out.reshape(B, S, H)
